# baseline (speedup 1.0000x reference)
import functools

import jax
import jax.numpy as jnp
from jax import lax
from jax.experimental import pallas as pl
from jax.experimental.pallas import tpu as pltpu
from jax.experimental.pallas import tpu_sc as plsc

D_MODEL = 64
SCALE = 8.0
NUM_CORES = 2
NUM_SUBCORES = 16
NUM_WORKERS = NUM_CORES * NUM_SUBCORES
SUB = 128
K = 1
CHUNK = SUB * K
LANES = 16


@functools.partial(jax.jit, static_argnames=("n",))
def _sc_embed(x2d, table2, n):
    per_w = n // NUM_WORKERS
    n_chunks = per_w // CHUNK
    n_pairs = n_chunks // 2
    idx_rows = per_w // SUB
    mesh = plsc.VectorSubcoreMesh(
        core_axis_name="c",
        subcore_axis_name="s",
        num_cores=NUM_CORES,
        num_subcores=NUM_SUBCORES,
    )

    @functools.partial(
        pl.kernel,
        mesh=mesh,
        out_type=jax.ShapeDtypeStruct((n, D_MODEL), jnp.float32),
        scratch_types=[
            pltpu.VMEM((idx_rows, SUB), jnp.int32),
            pltpu.VMEM((K, SUB), jnp.int32),
            pltpu.VMEM((K, SUB), jnp.int32),
            pltpu.VMEM((CHUNK, 2 * D_MODEL), jnp.float32),
            pltpu.VMEM((CHUNK, 2 * D_MODEL), jnp.float32),
            pltpu.VMEM((CHUNK, D_MODEL), jnp.float32),
            pltpu.VMEM((CHUNK, D_MODEL), jnp.float32),
            pltpu.SemaphoreType.DMA,
            pltpu.SemaphoreType.DMA,
            pltpu.SemaphoreType.DMA,
            pltpu.SemaphoreType.DMA,
        ],
    )
    def body(x_hbm, tab_hbm, out_hbm, idxb, pb0, pb1, gb0, gb1, ob0, ob1,
             sg0, sg1, so0, so1):
        wid = lax.axis_index("s") * NUM_CORES + lax.axis_index("c")
        base = wid * per_w
        base_row = pl.multiple_of(base // SUB, 8)
        pbs = (pb0, pb1)
        gbs = (gb0, gb1)
        obs = (ob0, ob1)
        gsems = (sg0, sg1)
        osems = (so0, so1)

        pltpu.sync_copy(x_hbm.at[pl.ds(base_row, idx_rows)], idxb)

        def prep_pairs(g, pb):
            for j in range(K):
                for c in range(SUB // LANES):
                    sl = pl.ds(c * LANES, LANES)
                    pb[j, sl] = lax.shift_right_logical(idxb[g * K + j, sl], 1)

        def fire_gathers(pb, gb, sem):
            for j in range(K):
                pltpu.async_copy(
                    tab_hbm.at[pb.at[j]],
                    gb.at[pl.ds(j * SUB, SUB)],
                    sem,
                )

        def wait_gathers(pb, gb, sem):
            for j in range(K):
                pltpu.make_async_copy(
                    tab_hbm.at[pb.at[j]],
                    gb.at[pl.ds(j * SUB, SUB)],
                    sem,
                ).wait()

        def out_slice(g):
            return out_hbm.at[
                pl.ds(pl.multiple_of(base + g * CHUNK, CHUNK), CHUNK)
            ]

        def out_copy_desc(ob, g, sem):
            return pltpu.make_async_copy(ob, out_slice(g), sem)

        def sel_scale(g, gb, ob):
            def grp_body(q, _):
                p0 = q * LANES
                fp = g * CHUNK + p0
                idxv = idxb[fp >> 7, pl.ds(fp & (SUB - 1), LANES)]
                hv = lax.shift_left(idxv & 1, 6)
                for i in range(LANES):
                    r = p0 + i
                    h = hv[i]
                    for c in range(D_MODEL // LANES):
                        ob[r, pl.ds(c * LANES, LANES)] = (
                            gb[r, pl.ds(h + c * LANES, LANES)] * SCALE
                        )
                return ()

            lax.fori_loop(0, CHUNK // LANES, grp_body, ())

        prep_pairs(0, pb0)
        fire_gathers(pb0, gb0, sg0)

        def pair_body(t, _):
            for phase in range(2):
                g = 2 * t + phase
                other = 1 - phase
                wait_gathers(pbs[phase], gbs[phase], gsems[phase])

                @pl.when(g < n_chunks - 1)
                def _():
                    prep_pairs(g + 1, pbs[other])
                    fire_gathers(pbs[other], gbs[other], gsems[other])

                @pl.when(g >= 2)
                def _():
                    out_copy_desc(obs[phase], g - 2, osems[phase]).wait()

                sel_scale(g, gbs[phase], obs[phase])
                out_copy_desc(obs[phase], g, osems[phase]).start()
            return ()

        lax.fori_loop(0, n_pairs, pair_body, ())
        out_copy_desc(ob0, n_chunks - 2, so0).wait()
        out_copy_desc(ob1, n_chunks - 1, so1).wait()

    return body(x2d, table2)


def kernel(x, table):
    b, s = x.shape
    n = b * s
    v = table.shape[0]
    x2d = x.reshape(n // SUB, SUB).astype(jnp.int32)
    table2 = table.reshape(v // 2, 2 * D_MODEL)
    out = _sc_embed(x2d, table2, n)
    return out.reshape(b, s, D_MODEL)
